# baseline (speedup 1.0000x reference)
import jax
import jax.numpy as jnp
from jax import lax
from jax.experimental import pallas as pl
from jax.experimental.pallas import tpu as pltpu
from jax.experimental.pallas import tpu_sc as plsc

_CH = 2000
_GS = 128


def _make_fused(n16, b):
    info = plsc.get_sparse_core_info()
    nc, ns = info.num_cores, info.num_subcores
    nw = nc * ns
    b_per_w = b // nw
    n_shots = b_per_w // _GS
    per_w_chunks = -(-n16 // (_CH * nw))
    if per_w_chunks % 2:
        per_w_chunks += 1
    mesh = plsc.VectorSubcoreMesh(core_axis_name="c", subcore_axis_name="s")

    @pl.kernel(
        mesh=mesh,
        out_type=(
            jax.ShapeDtypeStruct((2, b, 16), jnp.float32),
            jax.ShapeDtypeStruct((n16, 16), jnp.float32),
        ),
        scratch_types=[
            pltpu.VMEM((_GS,), jnp.int32),
            pltpu.VMEM((_GS,), jnp.int32),
            pltpu.VMEM((_GS, 16), jnp.float32),
            pltpu.VMEM((_CH, 16), jnp.float32),
            pltpu.VMEM((_CH, 16), jnp.float32),
            pltpu.SemaphoreType.DMA,
            pltpu.SemaphoreType.DMA,
            pltpu.SemaphoreType.DMA,
        ],
        compiler_params=pltpu.CompilerParams(use_tc_tiling_on_sc=False),
    )
    def fused_k(tbl_hbm, idx_hbm, out_hbm, reg_hbm, idx_v, row_v, got_v,
                buf_a, buf_b, sem_g, sem_a, sem_b):
        wid = lax.axis_index("s") * nc + lax.axis_index("c")

        base = wid * b_per_w

        @pl.loop(0, n_shots)
        def _shot(h):
            pltpu.sync_copy(idx_hbm.at[pl.ds(base + h * _GS, _GS)], idx_v)
            for par in range(2):

                @plsc.parallel_loop(0, _GS // 16, unroll=4)
                def _xform(i):
                    v = idx_v[pl.ds(i * 16, 16)]
                    row_v[pl.ds(i * 16, 16)] = v * 2 + par

                pltpu.async_copy(tbl_hbm.at[row_v], got_v, sem_g).wait()
                pltpu.sync_copy(
                    got_v, out_hbm.at[par, pl.ds(base + h * _GS, _GS)])

        def src(cj):
            return tbl_hbm.at[pl.ds((wid + cj * nw) * _CH, _CH)]

        def dst(cj):
            return reg_hbm.at[pl.ds((wid + cj * nw) * _CH, _CH)]

        def start(cj, buf, sem):
            @pl.when((wid + cj * nw) * _CH < n16)
            def _():
                pltpu.async_copy(src(cj), buf, sem)

        def finish(cj, buf, sem):
            @pl.when((wid + cj * nw) * _CH < n16)
            def _():
                pltpu.make_async_copy(src(cj), buf, sem).wait()

                @plsc.parallel_loop(0, _CH, unroll=8)
                def _row(i):
                    x = buf[i]
                    buf[i] = x * x

                pltpu.sync_copy(buf, dst(cj))

        start(0, buf_a, sem_a)

        @pl.loop(0, per_w_chunks, step=2)
        def _chunks(c):
            start(c + 1, buf_b, sem_b)
            finish(c, buf_a, sem_a)

            @pl.when(c + 2 < per_w_chunks)
            def _():
                start(c + 2, buf_a, sem_a)

            finish(c + 1, buf_b, sem_b)

    return fused_k


def kernel(indices, core_param):
    r1, n, r2 = core_param.shape
    b = indices.shape[0]
    n16 = (r1 * n * r2) // 16
    tbl = core_param.reshape(n16, 16)
    out3, reg2d = _make_fused(n16, b)(tbl, indices.astype(jnp.int32))
    out = jnp.transpose(out3, (1, 0, 2)).reshape(b, r1, r2)
    return out, reg2d.reshape(r1, n, r2)
